# Initial kernel scaffold; baseline (speedup 1.0000x reference)
import jax
import jax.numpy as jnp
from jax.experimental import pallas as pl


def kernel(input_ids, table):
    raise NotImplementedError("write your pallas kernel here")


# speedup vs baseline: 2.2163x; 2.2163x over previous
import functools
import jax
import jax.numpy as jnp
from jax import lax
from jax.experimental import pallas as pl
from jax.experimental.pallas import tpu as pltpu
from jax.experimental.pallas import tpu_sc as plsc

_NC = 2
_NS = 16
_NW = _NC * _NS


def _gather_call(table, idx_flat, n, d):
    b_per_w = n // _NW
    w = 80
    nch = b_per_w // w
    mesh = plsc.VectorSubcoreMesh(core_axis_name="c", subcore_axis_name="s")

    @functools.partial(
        pl.kernel,
        out_type=jax.ShapeDtypeStruct((n, d), table.dtype),
        mesh=mesh,
        scratch_types=[
            pltpu.VMEM((b_per_w,), jnp.int32),
            pltpu.VMEM((w, d), jnp.float32),
            pltpu.VMEM((w, d), jnp.float32),
            pltpu.SemaphoreType.DMA,
            pltpu.SemaphoreType.DMA,
        ],
    )
    def gather_kernel(tab_hbm, idx_hbm, out_hbm, idx_v, rows0, rows1, o0, o1):
        wid = lax.axis_index("s") * _NC + lax.axis_index("c")
        base = wid * b_per_w
        pltpu.sync_copy(idx_hbm.at[pl.ds(base, b_per_w)], idx_v)

        @pl.loop(0, nch, step=2)
        def _(kk):
            for bi, (rows, osem) in enumerate(((rows0, o0), (rows1, o1))):
                c = kk + bi

                @pl.when(kk > 0)
                def _():
                    pltpu.make_async_copy(
                        rows, out_hbm.at[pl.ds(base, w)], osem
                    ).wait()

                pltpu.sync_copy(tab_hbm.at[idx_v.at[pl.ds(c * w, w)]], rows)
                pltpu.async_copy(rows, out_hbm.at[pl.ds(base + c * w, w)], osem)

        for rows, osem in ((rows0, o0), (rows1, o1)):
            pltpu.make_async_copy(rows, out_hbm.at[pl.ds(base, w)], osem).wait()

    return gather_kernel(table, idx_flat)


def kernel(input_ids, table):
    b, l = input_ids.shape
    v, d = table.shape
    n = b * l
    idx_flat = input_ids.reshape(n).astype(jnp.int32)
    out = _gather_call(table, idx_flat, n, d)
    return out.reshape(b, l, d)
